# initial kernel scaffold (unmeasured)
import jax
import jax.numpy as jnp
from jax import lax
from jax.experimental import pallas as pl
from jax.experimental.pallas import tpu as pltpu

N_DEV = 8
B = 8
H = 8
D = 128
BS = 16
PAGES = 512
NB = 512
NK = PAGES * BS
NEG = -1e30
SCALE = D ** -0.5


def kernel(Q, K, V, bt, lens):
    def body(q_ref, k_hbm, v_hbm, bt_ref, lens_ref, out_ref,
             kbuf, vbuf, comm_acc, comm_ml,
             local_sems, acc_send, acc_recv, ml_send, ml_recv):
        my = lax.axis_index("i")
        base = my * PAGES

        barrier = pltpu.get_barrier_semaphore()
        for t in range(N_DEV):
            @pl.when(my != t)
            def _(t=t):
                pl.semaphore_signal(
                    barrier, inc=1,
                    device_id=(t,), device_id_type=pl.DeviceIdType.MESH,
                )

        def kv_copies(h):
            ck = pltpu.make_async_copy(
                k_hbm.at[:, :, pl.ds(h, 1), :], kbuf.at[h % 2],
                local_sems.at[h % 2, 0])
            cv = pltpu.make_async_copy(
                v_hbm.at[:, :, pl.ds(h, 1), :], vbuf.at[h % 2],
                local_sems.at[h % 2, 1])
            return ck, cv

        ck0, cv0 = kv_copies(0)
        ck0.start()
        cv0.start()

        btT = jnp.transpose(bt_ref[...])
        p_row = lax.broadcasted_iota(jnp.int32, (NB, PAGES), 1)
        j_col = lax.broadcasted_iota(jnp.int32, (NB, 1), 0)
        rows = []
        for b in range(B):
            col = btT[:, b:b + 1]
            valid = j_col < lens_ref[b]
            match = (col == base + p_row) & valid
            rows.append(jnp.sum(
                jnp.where(match, 1.0, 0.0).astype(jnp.float32),
                axis=0, keepdims=True))
        C = jnp.concatenate(rows, axis=0)
        k16 = lax.broadcasted_iota(jnp.int32, (PAGES, NK), 1) // BS
        p_col = lax.broadcasted_iota(jnp.int32, (PAGES, NK), 0)
        E = jnp.where(k16 == p_col, 1.0, 0.0).astype(jnp.float32)
        W = lax.dot_general(C, E, (((1,), (0,)), ((), ())),
                            preferred_element_type=jnp.float32)
        Wpos = W > 0

        q3 = q_ref[...].reshape(B, H, D)

        accs, ms, ls = [], [], []
        for h in range(H):
            ck, cv = kv_copies(h)
            if h + 1 < H:
                nk, nv = kv_copies(h + 1)
                nk.start()
                nv.start()
            ck.wait()
            cv.wait()
            k2 = kbuf[h % 2].reshape(NK, D)
            v2 = vbuf[h % 2].reshape(NK, D)
            q_h = q3[:, h, :]
            s = lax.dot_general(q_h, k2, (((1,), (1,)), ((), ())),
                                preferred_element_type=jnp.float32)
            s = jnp.where(Wpos, s * SCALE, NEG)
            m_h = jnp.max(s, axis=1, keepdims=True)
            p = jnp.exp(s - m_h) * W
            l_h = jnp.sum(p, axis=1, keepdims=True)
            acc = lax.dot_general(p, v2, (((1,), (0,)), ((), ())),
                                  preferred_element_type=jnp.float32)
            accs.append(acc[:, None, :])
            ms.append(m_h)
            ls.append(l_h)

        comm_acc[my] = jnp.concatenate(accs, axis=1)
        comm_ml[my, 0] = jnp.concatenate(ms, axis=1)
        comm_ml[my, 1] = jnp.concatenate(ls, axis=1)

        pl.semaphore_wait(barrier, N_DEV - 1)

        def acc_rdma(slot, t):
            return pltpu.make_async_remote_copy(
                src_ref=comm_acc.at[slot], dst_ref=comm_acc.at[slot],
                send_sem=acc_send.at[t], recv_sem=acc_recv.at[slot],
                device_id=(t,), device_id_type=pl.DeviceIdType.MESH)

        def ml_rdma(slot, t):
            return pltpu.make_async_remote_copy(
                src_ref=comm_ml.at[slot], dst_ref=comm_ml.at[slot],
                send_sem=ml_send.at[t], recv_sem=ml_recv.at[slot],
                device_id=(t,), device_id_type=pl.DeviceIdType.MESH)

        for t in range(N_DEV):
            @pl.when(my != t)
            def _(t=t):
                acc_rdma(my, t).start()
                ml_rdma(my, t).start()

        for s_ in range(N_DEV):
            @pl.when(my != s_)
            def _(s_=s_):
                acc_rdma(s_, s_).wait_recv()
                ml_rdma(s_, s_).wait_recv()

        for t in range(N_DEV):
            @pl.when(my != t)
            def _(t=t):
                acc_rdma(my, t).wait_send()
                ml_rdma(my, t).wait_send()

        A = comm_acc[...]
        m_all = comm_ml[...][:, 0]
        l_all = comm_ml[...][:, 1]
        M = jnp.max(m_all, axis=0, keepdims=True)
        alpha = jnp.exp(m_all - M)
        num = jnp.sum(A * alpha[..., None], axis=0)
        den = jnp.sum(l_all * alpha, axis=0)
        out = num / den[..., None]
        out_ref[...] = out[:, None, :, :]

    return pl.pallas_call(
        body,
        out_shape=jax.ShapeDtypeStruct((B, 1, H, D), jnp.float32),
        in_specs=[
            pl.BlockSpec(memory_space=pltpu.VMEM),
            pl.BlockSpec(memory_space=pltpu.ANY),
            pl.BlockSpec(memory_space=pltpu.ANY),
            pl.BlockSpec(memory_space=pltpu.VMEM),
            pl.BlockSpec(memory_space=pltpu.SMEM),
        ],
        out_specs=pl.BlockSpec(memory_space=pltpu.VMEM),
        scratch_shapes=[
            pltpu.VMEM((2, PAGES, BS, 1, D), jnp.float32),
            pltpu.VMEM((2, PAGES, BS, 1, D), jnp.float32),
            pltpu.VMEM((N_DEV, B, H, D), jnp.float32),
            pltpu.VMEM((N_DEV, 2, B, H), jnp.float32),
            pltpu.SemaphoreType.DMA((2, 2)),
            pltpu.SemaphoreType.DMA((N_DEV,)),
            pltpu.SemaphoreType.DMA((N_DEV,)),
            pltpu.SemaphoreType.DMA((N_DEV,)),
            pltpu.SemaphoreType.DMA((N_DEV,)),
        ],
        compiler_params=pltpu.CompilerParams(collective_id=0),
    )(Q, K, V, bt, lens)


# baseline (device time: 60672 ns/iter reference)
import jax
import jax.numpy as jnp
from jax import lax
from jax.experimental import pallas as pl
from jax.experimental.pallas import tpu as pltpu

N_DEV = 8
B = 8
H = 8
D = 128
BS = 16
PAGES = 512
NB = 512
NK = PAGES * BS
CP = 128
C = PAGES // CP
CK = CP * BS
NSTEP = H * C
NEG = -1e30
SCALE = D ** -0.5


def kernel(Q, K, V, bt, lens):
    qT = Q.reshape(B, H, D).transpose(1, 0, 2)

    def body(q_ref, k_hbm, v_hbm, bt_ref, lens_ref, out_ref,
             w_ref, m_run, l_run, acc_run, kbuf, vbuf, comm_acc, comm_ml,
             ksem, vsem, acc_send, acc_recv, ml_send, ml_recv):
        h = pl.program_id(0)
        c = pl.program_id(1)
        t = h * C + c
        my = lax.axis_index("i")
        base = my * PAGES
        barrier = pltpu.get_barrier_semaphore()

        def kv_dma(hh, cc, slot):
            ck = pltpu.make_async_copy(
                k_hbm.at[pl.ds(cc * CP, CP), :, pl.ds(hh, 1), :],
                kbuf.at[slot], ksem.at[slot])
            cv = pltpu.make_async_copy(
                v_hbm.at[pl.ds(cc * CP, CP), :, pl.ds(hh, 1), :],
                vbuf.at[slot], vsem.at[slot])
            return ck, cv

        @pl.when(t == 0)
        def _first():
            ck, cv = kv_dma(0, 0, 0)
            ck.start()
            cv.start()
            for tgt in range(N_DEV):
                @pl.when(my != tgt)
                def _(tgt=tgt):
                    pl.semaphore_signal(
                        barrier, inc=1,
                        device_id=(tgt,), device_id_type=pl.DeviceIdType.MESH,
                    )
            btT = jnp.transpose(bt_ref[...])
            p_row = lax.broadcasted_iota(jnp.int32, (NB, PAGES), 1)
            j_col = lax.broadcasted_iota(jnp.int32, (NB, 1), 0)
            rows = []
            for b in range(B):
                col = btT[:, b:b + 1]
                valid = j_col < lens_ref[b]
                match = (col == base + p_row) & valid
                rows.append(jnp.sum(
                    jnp.where(match, 1.0, 0.0).astype(jnp.float32),
                    axis=0, keepdims=True))
            cnt = jnp.concatenate(rows, axis=0)
            k16 = lax.broadcasted_iota(jnp.int32, (PAGES, NK), 1) // BS
            p_col = lax.broadcasted_iota(jnp.int32, (PAGES, NK), 0)
            expand = jnp.where(k16 == p_col, 1.0, 0.0).astype(jnp.float32)
            w_ref[...] = lax.dot_general(
                cnt, expand, (((1,), (0,)), ((), ())),
                preferred_element_type=jnp.float32)

        nxt = t + 1

        @pl.when(nxt < NSTEP)
        def _prefetch():
            ck, cv = kv_dma(nxt // C, lax.rem(nxt, C), lax.rem(nxt, 2))
            ck.start()
            cv.start()

        @pl.when(c == 0)
        def _head_init():
            m_run[...] = jnp.full((B, 1), NEG, jnp.float32)
            l_run[...] = jnp.zeros((B, 1), jnp.float32)
            acc_run[...] = jnp.zeros((B, D), jnp.float32)

        slot = lax.rem(t, 2)
        ck, cv = kv_dma(h, c, slot)
        ck.wait()
        cv.wait()
        k2 = kbuf[slot].reshape(CK, D)
        v2 = vbuf[slot].reshape(CK, D)
        q_h = q_ref[h].reshape(B, D)
        wc = w_ref[:, pl.ds(c * CK, CK)]
        s = lax.dot_general(q_h, k2, (((1,), (1,)), ((), ())),
                            preferred_element_type=jnp.float32)
        s = jnp.where(wc > 0, s * SCALE, NEG)
        m_c = jnp.max(s, axis=1, keepdims=True)
        m_new = jnp.maximum(m_run[...], m_c)
        alpha = jnp.exp(m_run[...] - m_new)
        p = jnp.exp(s - m_new) * wc
        l_run[...] = l_run[...] * alpha + jnp.sum(p, axis=1, keepdims=True)
        acc_run[...] = acc_run[...] * alpha + lax.dot_general(
            p, v2, (((1,), (0,)), ((), ())),
            preferred_element_type=jnp.float32)
        m_run[...] = m_new

        @pl.when(c == C - 1)
        def _head_done():
            comm_acc[my, h] = acc_run[...]
            comm_ml[my, h, 0] = m_run[...]
            comm_ml[my, h, 1] = l_run[...]

        def acc_rdma(slot_, tgt):
            return pltpu.make_async_remote_copy(
                src_ref=comm_acc.at[slot_], dst_ref=comm_acc.at[slot_],
                send_sem=acc_send.at[tgt], recv_sem=acc_recv.at[slot_],
                device_id=(tgt,), device_id_type=pl.DeviceIdType.MESH)

        def ml_rdma(slot_, tgt):
            return pltpu.make_async_remote_copy(
                src_ref=comm_ml.at[slot_], dst_ref=comm_ml.at[slot_],
                send_sem=ml_send.at[tgt], recv_sem=ml_recv.at[slot_],
                device_id=(tgt,), device_id_type=pl.DeviceIdType.MESH)

        @pl.when(t == NSTEP - 1)
        def _last():
            pl.semaphore_wait(barrier, N_DEV - 1)
            for tgt in range(N_DEV):
                @pl.when(my != tgt)
                def _(tgt=tgt):
                    acc_rdma(my, tgt).start()
                    ml_rdma(my, tgt).start()
            for src in range(N_DEV):
                @pl.when(my != src)
                def _(src=src):
                    acc_rdma(src, src).wait_recv()
                    ml_rdma(src, src).wait_recv()
            for tgt in range(N_DEV):
                @pl.when(my != tgt)
                def _(tgt=tgt):
                    acc_rdma(my, tgt).wait_send()
                    ml_rdma(my, tgt).wait_send()

            A = comm_acc[...]
            ml = comm_ml[...]
            m_all = ml[:, :, 0, :, 0]
            l_all = ml[:, :, 1, :, 0]
            M = jnp.max(m_all, axis=0, keepdims=True)
            w_dev = jnp.exp(m_all - M)
            num = jnp.sum(A * w_dev[..., None], axis=0)
            den = jnp.sum(l_all * w_dev, axis=0)
            out = (num / den[..., None]).transpose(1, 0, 2)
            out_ref[...] = out[:, None, :, :]

    return pl.pallas_call(
        body,
        grid=(H, C),
        out_shape=jax.ShapeDtypeStruct((B, 1, H, D), jnp.float32),
        in_specs=[
            pl.BlockSpec(memory_space=pltpu.MemorySpace.VMEM),
            pl.BlockSpec(memory_space=pltpu.MemorySpace.HBM),
            pl.BlockSpec(memory_space=pltpu.MemorySpace.HBM),
            pl.BlockSpec(memory_space=pltpu.MemorySpace.VMEM),
            pl.BlockSpec(memory_space=pltpu.SMEM),
        ],
        out_specs=pl.BlockSpec(memory_space=pltpu.MemorySpace.VMEM),
        scratch_shapes=[
            pltpu.VMEM((B, NK), jnp.float32),
            pltpu.VMEM((B, 1), jnp.float32),
            pltpu.VMEM((B, 1), jnp.float32),
            pltpu.VMEM((B, D), jnp.float32),
            pltpu.VMEM((2, CP, BS, 1, D), jnp.float32),
            pltpu.VMEM((2, CP, BS, 1, D), jnp.float32),
            pltpu.VMEM((N_DEV, H, B, D), jnp.float32),
            pltpu.VMEM((N_DEV, H, 2, B, 1), jnp.float32),
            pltpu.SemaphoreType.DMA((2,)),
            pltpu.SemaphoreType.DMA((2,)),
            pltpu.SemaphoreType.DMA((N_DEV,)),
            pltpu.SemaphoreType.DMA((N_DEV,)),
            pltpu.SemaphoreType.DMA((N_DEV,)),
            pltpu.SemaphoreType.DMA((N_DEV,)),
        ],
        compiler_params=pltpu.CompilerParams(
            collective_id=0,
            dimension_semantics=("arbitrary", "arbitrary"),
        ),
    )(qT, K, V, bt, lens)


# device time: 48054 ns/iter; 1.2626x vs baseline; 1.2626x over previous
import jax
import jax.numpy as jnp
from jax import lax
from jax.experimental import pallas as pl
from jax.experimental.pallas import tpu as pltpu

N_DEV = 8
B = 8
H = 8
D = 128
BS = 16
PAGES = 512
NB = 512
NK = PAGES * BS
CP = 512
C = PAGES // CP
CK = CP * BS
NSTEP = H * C
NEG = -1e30
SCALE = D ** -0.5


def kernel(Q, K, V, bt, lens):
    qT = Q.reshape(B, H, D).transpose(1, 0, 2)

    def body(q_ref, k_hbm, v_hbm, bt_ref, lens_ref, out_ref,
             w_ref, m_run, l_run, acc_run, kbuf, vbuf, comm_acc, comm_ml,
             ksem, vsem, acc_send, acc_recv, ml_send, ml_recv):
        h = pl.program_id(0)
        c = pl.program_id(1)
        t = h * C + c
        my = lax.axis_index("i")
        base = my * PAGES
        barrier = pltpu.get_barrier_semaphore()

        def kv_dma(hh, cc, slot):
            ck = pltpu.make_async_copy(
                k_hbm.at[pl.ds(cc * CP, CP), :, pl.ds(hh, 1), :],
                kbuf.at[slot], ksem.at[slot])
            cv = pltpu.make_async_copy(
                v_hbm.at[pl.ds(cc * CP, CP), :, pl.ds(hh, 1), :],
                vbuf.at[slot], vsem.at[slot])
            return ck, cv

        @pl.when(t == 0)
        def _first():
            ck, cv = kv_dma(0, 0, 0)
            ck.start()
            cv.start()
            for tgt in range(N_DEV):
                @pl.when(my != tgt)
                def _(tgt=tgt):
                    pl.semaphore_signal(
                        barrier, inc=1,
                        device_id=(tgt,), device_id_type=pl.DeviceIdType.MESH,
                    )
            btT = jnp.transpose(bt_ref[...])
            p_row = lax.broadcasted_iota(jnp.int32, (NB, PAGES), 1)
            j_col = lax.broadcasted_iota(jnp.int32, (NB, 1), 0)
            rows = []
            for b in range(B):
                col = btT[:, b:b + 1]
                valid = j_col < lens_ref[b]
                match = (col == base + p_row) & valid
                rows.append(jnp.sum(
                    jnp.where(match, 1.0, 0.0).astype(jnp.float32),
                    axis=0, keepdims=True))
            cnt = jnp.concatenate(rows, axis=0)
            k16 = lax.broadcasted_iota(jnp.int32, (PAGES, NK), 1) // BS
            p_col = lax.broadcasted_iota(jnp.int32, (PAGES, NK), 0)
            expand = jnp.where(k16 == p_col, 1.0, 0.0).astype(jnp.float32)
            w_ref[...] = lax.dot_general(
                cnt, expand, (((1,), (0,)), ((), ())),
                preferred_element_type=jnp.float32)

        nxt = t + 1

        @pl.when(nxt < NSTEP)
        def _prefetch():
            ck, cv = kv_dma(nxt // C, lax.rem(nxt, C), lax.rem(nxt, 2))
            ck.start()
            cv.start()

        @pl.when(c == 0)
        def _head_init():
            m_run[...] = jnp.full((B, 1), NEG, jnp.float32)
            l_run[...] = jnp.zeros((B, 1), jnp.float32)
            acc_run[...] = jnp.zeros((B, D), jnp.float32)

        slot = lax.rem(t, 2)
        ck, cv = kv_dma(h, c, slot)
        ck.wait()
        cv.wait()
        k2 = kbuf[slot].reshape(CK, D)
        v2 = vbuf[slot].reshape(CK, D)
        q_h = q_ref[h].reshape(B, D)
        wc = w_ref[:, pl.ds(c * CK, CK)]
        s = lax.dot_general(q_h, k2, (((1,), (1,)), ((), ())),
                            preferred_element_type=jnp.float32)
        s = jnp.where(wc > 0, s * SCALE, NEG)
        m_c = jnp.max(s, axis=1, keepdims=True)
        m_new = jnp.maximum(m_run[...], m_c)
        alpha = jnp.exp(m_run[...] - m_new)
        p = jnp.exp(s - m_new) * wc
        l_run[...] = l_run[...] * alpha + jnp.sum(p, axis=1, keepdims=True)
        acc_run[...] = acc_run[...] * alpha + lax.dot_general(
            p, v2, (((1,), (0,)), ((), ())),
            preferred_element_type=jnp.float32)
        m_run[...] = m_new

        @pl.when(c == C - 1)
        def _head_done():
            comm_acc[my, h] = acc_run[...]
            comm_ml[my, h, 0] = m_run[...]
            comm_ml[my, h, 1] = l_run[...]

        def acc_rdma(slot_, tgt):
            return pltpu.make_async_remote_copy(
                src_ref=comm_acc.at[slot_], dst_ref=comm_acc.at[slot_],
                send_sem=acc_send.at[tgt], recv_sem=acc_recv.at[slot_],
                device_id=(tgt,), device_id_type=pl.DeviceIdType.MESH)

        def ml_rdma(slot_, tgt):
            return pltpu.make_async_remote_copy(
                src_ref=comm_ml.at[slot_], dst_ref=comm_ml.at[slot_],
                send_sem=ml_send.at[tgt], recv_sem=ml_recv.at[slot_],
                device_id=(tgt,), device_id_type=pl.DeviceIdType.MESH)

        @pl.when(t == NSTEP - 1)
        def _last():
            pl.semaphore_wait(barrier, N_DEV - 1)
            for tgt in range(N_DEV):
                @pl.when(my != tgt)
                def _(tgt=tgt):
                    acc_rdma(my, tgt).start()
                    ml_rdma(my, tgt).start()
            for src in range(N_DEV):
                @pl.when(my != src)
                def _(src=src):
                    acc_rdma(src, src).wait_recv()
                    ml_rdma(src, src).wait_recv()
            for tgt in range(N_DEV):
                @pl.when(my != tgt)
                def _(tgt=tgt):
                    acc_rdma(my, tgt).wait_send()
                    ml_rdma(my, tgt).wait_send()

            A = comm_acc[...]
            ml = comm_ml[...]
            m_all = ml[:, :, 0, :, 0]
            l_all = ml[:, :, 1, :, 0]
            M = jnp.max(m_all, axis=0, keepdims=True)
            w_dev = jnp.exp(m_all - M)
            num = jnp.sum(A * w_dev[..., None], axis=0)
            den = jnp.sum(l_all * w_dev, axis=0)
            out = (num / den[..., None]).transpose(1, 0, 2)
            out_ref[...] = out[:, None, :, :]

    return pl.pallas_call(
        body,
        grid=(H, C),
        out_shape=jax.ShapeDtypeStruct((B, 1, H, D), jnp.float32),
        in_specs=[
            pl.BlockSpec(memory_space=pltpu.MemorySpace.VMEM),
            pl.BlockSpec(memory_space=pltpu.MemorySpace.HBM),
            pl.BlockSpec(memory_space=pltpu.MemorySpace.HBM),
            pl.BlockSpec(memory_space=pltpu.MemorySpace.VMEM),
            pl.BlockSpec(memory_space=pltpu.SMEM),
        ],
        out_specs=pl.BlockSpec(memory_space=pltpu.MemorySpace.VMEM),
        scratch_shapes=[
            pltpu.VMEM((B, NK), jnp.float32),
            pltpu.VMEM((B, 1), jnp.float32),
            pltpu.VMEM((B, 1), jnp.float32),
            pltpu.VMEM((B, D), jnp.float32),
            pltpu.VMEM((2, CP, BS, 1, D), jnp.float32),
            pltpu.VMEM((2, CP, BS, 1, D), jnp.float32),
            pltpu.VMEM((N_DEV, H, B, D), jnp.float32),
            pltpu.VMEM((N_DEV, H, 2, B, 1), jnp.float32),
            pltpu.SemaphoreType.DMA((2,)),
            pltpu.SemaphoreType.DMA((2,)),
            pltpu.SemaphoreType.DMA((N_DEV,)),
            pltpu.SemaphoreType.DMA((N_DEV,)),
            pltpu.SemaphoreType.DMA((N_DEV,)),
            pltpu.SemaphoreType.DMA((N_DEV,)),
        ],
        compiler_params=pltpu.CompilerParams(
            collective_id=0,
            dimension_semantics=("arbitrary", "arbitrary"),
        ),
    )(qT, K, V, bt, lens)


# device time: 44716 ns/iter; 1.3568x vs baseline; 1.0746x over previous
import jax
import jax.numpy as jnp
from jax import lax
from jax.experimental import pallas as pl
from jax.experimental.pallas import tpu as pltpu

N_DEV = 8
B = 8
H = 8
D = 128
BS = 16
PAGES = 512
NB = 512
NK = PAGES * BS
CP = 64
C = PAGES // CP
CK = CP * BS
NEG = -1e30
SCALE = D ** -0.5


def kernel(Q, K, V, bt, lens):
    qT = Q.reshape(B, H, D).transpose(1, 0, 2)

    def body(q_ref, k_hbm, v_hbm, bt_ref, lens_ref, out_ref,
             w_ref, m_run, l_run, acc_run, kbuf, vbuf, comm_acc, comm_ml,
             ksem, vsem, acc_send, acc_recv, ml_send, ml_recv):
        c = pl.program_id(0)
        my = lax.axis_index("i")
        base = my * PAGES
        barrier = pltpu.get_barrier_semaphore()

        def kv_dma(cc, slot):
            ck = pltpu.make_async_copy(
                k_hbm.at[pl.ds(cc * CP, CP)], kbuf.at[slot], ksem.at[slot])
            cv = pltpu.make_async_copy(
                v_hbm.at[pl.ds(cc * CP, CP)], vbuf.at[slot], vsem.at[slot])
            return ck, cv

        @pl.when(c == 0)
        def _first():
            ck, cv = kv_dma(0, 0)
            ck.start()
            cv.start()
            for tgt in range(N_DEV):
                @pl.when(my != tgt)
                def _(tgt=tgt):
                    pl.semaphore_signal(
                        barrier, inc=1,
                        device_id=(tgt,), device_id_type=pl.DeviceIdType.MESH,
                    )
            m_run[...] = jnp.full((H, B, 1), NEG, jnp.float32)
            l_run[...] = jnp.zeros((H, B, 1), jnp.float32)
            acc_run[...] = jnp.zeros((H, B, D), jnp.float32)
            btT = jnp.transpose(bt_ref[...])
            p_row = lax.broadcasted_iota(jnp.int32, (NB, PAGES), 1)
            j_col = lax.broadcasted_iota(jnp.int32, (NB, 1), 0)
            rows = []
            for b in range(B):
                col = btT[:, b:b + 1]
                valid = j_col < lens_ref[b]
                match = (col == base + p_row) & valid
                rows.append(jnp.sum(
                    jnp.where(match, 1.0, 0.0).astype(jnp.float32),
                    axis=0, keepdims=True))
            cnt = jnp.concatenate(rows, axis=0)
            k16 = lax.broadcasted_iota(jnp.int32, (PAGES, NK), 1) // BS
            p_col = lax.broadcasted_iota(jnp.int32, (PAGES, NK), 0)
            expand = jnp.where(k16 == p_col, 1.0, 0.0).astype(jnp.float32)
            w_ref[...] = lax.dot_general(
                cnt, expand, (((1,), (0,)), ((), ())),
                preferred_element_type=jnp.float32)

        @pl.when(c + 1 < C)
        def _prefetch():
            ck, cv = kv_dma(c + 1, lax.rem(c + 1, 2))
            ck.start()
            cv.start()

        slot = lax.rem(c, 2)
        ck, cv = kv_dma(c, slot)
        ck.wait()
        cv.wait()
        kflat = kbuf[slot].reshape(CK, H * D)
        vflat = vbuf[slot].reshape(CK, H * D)
        q3 = q_ref[...]
        wc = w_ref[:, pl.ds(c * CK, CK)]
        wpos = wc > 0
        for h in range(H):
            k2 = kflat[:, h * D:(h + 1) * D]
            v2 = vflat[:, h * D:(h + 1) * D]
            s = lax.dot_general(q3[h], k2, (((1,), (1,)), ((), ())),
                                preferred_element_type=jnp.float32)
            s = jnp.where(wpos, s * SCALE, NEG)
            m_c = jnp.max(s, axis=1, keepdims=True)
            m_new = jnp.maximum(m_run[h], m_c)
            alpha = jnp.exp(m_run[h] - m_new)
            p = jnp.exp(s - m_new) * wc
            l_run[h] = l_run[h] * alpha + jnp.sum(p, axis=1, keepdims=True)
            acc_run[h] = acc_run[h] * alpha + lax.dot_general(
                p, v2, (((1,), (0,)), ((), ())),
                preferred_element_type=jnp.float32)
            m_run[h] = m_new

        def acc_rdma(slot_, tgt):
            return pltpu.make_async_remote_copy(
                src_ref=comm_acc.at[slot_], dst_ref=comm_acc.at[slot_],
                send_sem=acc_send.at[tgt], recv_sem=acc_recv.at[slot_],
                device_id=(tgt,), device_id_type=pl.DeviceIdType.MESH)

        def ml_rdma(slot_, tgt):
            return pltpu.make_async_remote_copy(
                src_ref=comm_ml.at[slot_], dst_ref=comm_ml.at[slot_],
                send_sem=ml_send.at[tgt], recv_sem=ml_recv.at[slot_],
                device_id=(tgt,), device_id_type=pl.DeviceIdType.MESH)

        @pl.when(c == C - 1)
        def _last():
            comm_acc[my] = acc_run[...]
            comm_ml[my, :, 0] = m_run[...]
            comm_ml[my, :, 1] = l_run[...]
            pl.semaphore_wait(barrier, N_DEV - 1)
            for tgt in range(N_DEV):
                @pl.when(my != tgt)
                def _(tgt=tgt):
                    acc_rdma(my, tgt).start()
                    ml_rdma(my, tgt).start()
            for src in range(N_DEV):
                @pl.when(my != src)
                def _(src=src):
                    acc_rdma(src, src).wait_recv()
                    ml_rdma(src, src).wait_recv()
            for tgt in range(N_DEV):
                @pl.when(my != tgt)
                def _(tgt=tgt):
                    acc_rdma(my, tgt).wait_send()
                    ml_rdma(my, tgt).wait_send()

            A = comm_acc[...]
            ml = comm_ml[...]
            m_all = ml[:, :, 0, :, 0]
            l_all = ml[:, :, 1, :, 0]
            M = jnp.max(m_all, axis=0, keepdims=True)
            w_dev = jnp.exp(m_all - M)
            num = jnp.sum(A * w_dev[..., None], axis=0)
            den = jnp.sum(l_all * w_dev, axis=0)
            out = (num / den[..., None]).transpose(1, 0, 2)
            out_ref[...] = out[:, None, :, :]

    return pl.pallas_call(
        body,
        grid=(C,),
        out_shape=jax.ShapeDtypeStruct((B, 1, H, D), jnp.float32),
        in_specs=[
            pl.BlockSpec(memory_space=pltpu.MemorySpace.VMEM),
            pl.BlockSpec(memory_space=pltpu.MemorySpace.HBM),
            pl.BlockSpec(memory_space=pltpu.MemorySpace.HBM),
            pl.BlockSpec(memory_space=pltpu.MemorySpace.VMEM),
            pl.BlockSpec(memory_space=pltpu.SMEM),
        ],
        out_specs=pl.BlockSpec(memory_space=pltpu.MemorySpace.VMEM),
        scratch_shapes=[
            pltpu.VMEM((B, NK), jnp.float32),
            pltpu.VMEM((H, B, 1), jnp.float32),
            pltpu.VMEM((H, B, 1), jnp.float32),
            pltpu.VMEM((H, B, D), jnp.float32),
            pltpu.VMEM((2, CP, BS, H, D), jnp.float32),
            pltpu.VMEM((2, CP, BS, H, D), jnp.float32),
            pltpu.VMEM((N_DEV, H, B, D), jnp.float32),
            pltpu.VMEM((N_DEV, H, 2, B, 1), jnp.float32),
            pltpu.SemaphoreType.DMA((2,)),
            pltpu.SemaphoreType.DMA((2,)),
            pltpu.SemaphoreType.DMA((N_DEV,)),
            pltpu.SemaphoreType.DMA((N_DEV,)),
            pltpu.SemaphoreType.DMA((N_DEV,)),
            pltpu.SemaphoreType.DMA((N_DEV,)),
        ],
        compiler_params=pltpu.CompilerParams(
            collective_id=0,
            dimension_semantics=("arbitrary",),
        ),
    )(qT, K, V, bt, lens)


# device time: 36173 ns/iter; 1.6773x vs baseline; 1.2362x over previous
import jax
import jax.numpy as jnp
from jax import lax
from jax.experimental import pallas as pl
from jax.experimental.pallas import tpu as pltpu

N_DEV = 8
B = 8
H = 8
D = 128
BS = 16
PAGES = 512
NB = 512
NK = PAGES * BS
CP = 64
C = PAGES // CP
CK = CP * BS
NEG = -1e30
SCALE = D ** -0.5


def kernel(Q, K, V, bt, lens):
    qT = Q.reshape(B, H, D).transpose(1, 0, 2)

    def body(q_ref, k_hbm, v_hbm, bt_ref, lens_ref, out_ref,
             w_ref, m_run, l_run, acc_run, kbuf, vbuf, comm_acc, comm_ml,
             ksem, vsem, acc_send, acc_recv, ml_send, ml_recv):
        c = pl.program_id(0)
        my = lax.axis_index("i")
        base = my * PAGES
        barrier = pltpu.get_barrier_semaphore()

        def kv_dma(cc, slot):
            ck = pltpu.make_async_copy(
                k_hbm.at[pl.ds(cc * CP, CP)], kbuf.at[slot], ksem.at[slot])
            cv = pltpu.make_async_copy(
                v_hbm.at[pl.ds(cc * CP, CP)], vbuf.at[slot], vsem.at[slot])
            return ck, cv

        @pl.when(c == 0)
        def _first():
            ck, cv = kv_dma(0, 0)
            ck.start()
            cv.start()
            for tgt in range(N_DEV):
                @pl.when(my != tgt)
                def _(tgt=tgt):
                    pl.semaphore_signal(
                        barrier, inc=1,
                        device_id=(tgt,), device_id_type=pl.DeviceIdType.MESH,
                    )
            m_run[...] = jnp.full((H * B, 1), NEG, jnp.float32)
            l_run[...] = jnp.zeros((H * B, 1), jnp.float32)
            acc_run[...] = jnp.zeros((H * B, D), jnp.float32)
            btT = jnp.transpose(bt_ref[...])
            p_row = lax.broadcasted_iota(jnp.int32, (NB, PAGES), 1)
            j_col = lax.broadcasted_iota(jnp.int32, (NB, 1), 0)
            rows = []
            for b in range(B):
                col = btT[:, b:b + 1]
                valid = j_col < lens_ref[b]
                match = (col == base + p_row) & valid
                rows.append(jnp.sum(
                    jnp.where(match, 1.0, 0.0).astype(jnp.float32),
                    axis=0, keepdims=True))
            cnt = jnp.concatenate(rows, axis=0)
            k16 = lax.broadcasted_iota(jnp.int32, (PAGES, NK), 1) // BS
            p_col = lax.broadcasted_iota(jnp.int32, (PAGES, NK), 0)
            expand = jnp.where(k16 == p_col, 1.0, 0.0).astype(jnp.bfloat16)
            w_ref[...] = lax.dot_general(
                cnt.astype(jnp.bfloat16), expand, (((1,), (0,)), ((), ())),
                preferred_element_type=jnp.float32)

        @pl.when(c + 1 < C)
        def _prefetch():
            ck, cv = kv_dma(c + 1, lax.rem(c + 1, 2))
            ck.start()
            cv.start()

        slot = lax.rem(c, 2)
        ck, cv = kv_dma(c, slot)
        ck.wait()
        cv.wait()
        kflat = kbuf[slot].reshape(CK, H * D)
        vflat = vbuf[slot].reshape(CK, H * D)
        q3 = q_ref[...]
        wc = w_ref[:, pl.ds(c * CK, CK)]
        s_all = jnp.concatenate(
            [lax.dot_general(q3[h], kflat[:, h * D:(h + 1) * D],
                             (((1,), (1,)), ((), ())),
                             preferred_element_type=jnp.float32)
             for h in range(H)], axis=0)
        wt = jnp.broadcast_to(wc[None], (H, B, CK)).reshape(H * B, CK)
        s_all = jnp.where(wt > 0, s_all * SCALE, NEG)
        m_c = jnp.max(s_all, axis=1, keepdims=True)
        m_new = jnp.maximum(m_run[...], m_c)
        alpha = jnp.exp(m_run[...] - m_new)
        p = jnp.exp(s_all - m_new) * wt
        pv = jnp.concatenate(
            [lax.dot_general(p[h * B:(h + 1) * B],
                             vflat[:, h * D:(h + 1) * D],
                             (((1,), (0,)), ((), ())),
                             preferred_element_type=jnp.float32)
             for h in range(H)], axis=0)
        l_run[...] = l_run[...] * alpha + jnp.sum(p, axis=1, keepdims=True)
        acc_run[...] = acc_run[...] * alpha + pv
        m_run[...] = m_new

        def acc_rdma(slot_, tgt):
            return pltpu.make_async_remote_copy(
                src_ref=comm_acc.at[slot_], dst_ref=comm_acc.at[slot_],
                send_sem=acc_send.at[tgt], recv_sem=acc_recv.at[slot_],
                device_id=(tgt,), device_id_type=pl.DeviceIdType.MESH)

        def ml_rdma(slot_, tgt):
            return pltpu.make_async_remote_copy(
                src_ref=comm_ml.at[slot_], dst_ref=comm_ml.at[slot_],
                send_sem=ml_send.at[tgt], recv_sem=ml_recv.at[slot_],
                device_id=(tgt,), device_id_type=pl.DeviceIdType.MESH)

        @pl.when(c == C - 1)
        def _last():
            comm_acc[my] = acc_run[...].reshape(H, B, D)
            comm_ml[my, :, 0] = m_run[...].reshape(H, B, 1)
            comm_ml[my, :, 1] = l_run[...].reshape(H, B, 1)
            pl.semaphore_wait(barrier, N_DEV - 1)
            for tgt in range(N_DEV):
                @pl.when(my != tgt)
                def _(tgt=tgt):
                    acc_rdma(my, tgt).start()
                    ml_rdma(my, tgt).start()
            for src in range(N_DEV):
                @pl.when(my != src)
                def _(src=src):
                    acc_rdma(src, src).wait_recv()
                    ml_rdma(src, src).wait_recv()
            for tgt in range(N_DEV):
                @pl.when(my != tgt)
                def _(tgt=tgt):
                    acc_rdma(my, tgt).wait_send()
                    ml_rdma(my, tgt).wait_send()

            A = comm_acc[...]
            ml = comm_ml[...]
            m_all = ml[:, :, 0, :, 0]
            l_all = ml[:, :, 1, :, 0]
            M = jnp.max(m_all, axis=0, keepdims=True)
            w_dev = jnp.exp(m_all - M)
            num = jnp.sum(A * w_dev[..., None], axis=0)
            den = jnp.sum(l_all * w_dev, axis=0)
            out = (num / den[..., None]).transpose(1, 0, 2)
            out_ref[...] = out[:, None, :, :]

    return pl.pallas_call(
        body,
        grid=(C,),
        out_shape=jax.ShapeDtypeStruct((B, 1, H, D), jnp.float32),
        in_specs=[
            pl.BlockSpec(memory_space=pltpu.MemorySpace.VMEM),
            pl.BlockSpec(memory_space=pltpu.MemorySpace.HBM),
            pl.BlockSpec(memory_space=pltpu.MemorySpace.HBM),
            pl.BlockSpec(memory_space=pltpu.MemorySpace.VMEM),
            pl.BlockSpec(memory_space=pltpu.SMEM),
        ],
        out_specs=pl.BlockSpec(memory_space=pltpu.MemorySpace.VMEM),
        scratch_shapes=[
            pltpu.VMEM((B, NK), jnp.float32),
            pltpu.VMEM((H * B, 1), jnp.float32),
            pltpu.VMEM((H * B, 1), jnp.float32),
            pltpu.VMEM((H * B, D), jnp.float32),
            pltpu.VMEM((2, CP, BS, H, D), jnp.float32),
            pltpu.VMEM((2, CP, BS, H, D), jnp.float32),
            pltpu.VMEM((N_DEV, H, B, D), jnp.float32),
            pltpu.VMEM((N_DEV, H, 2, B, 1), jnp.float32),
            pltpu.SemaphoreType.DMA((2,)),
            pltpu.SemaphoreType.DMA((2,)),
            pltpu.SemaphoreType.DMA((N_DEV,)),
            pltpu.SemaphoreType.DMA((N_DEV,)),
            pltpu.SemaphoreType.DMA((N_DEV,)),
            pltpu.SemaphoreType.DMA((N_DEV,)),
        ],
        compiler_params=pltpu.CompilerParams(
            collective_id=0,
            dimension_semantics=("arbitrary",),
        ),
    )(qT, K, V, bt, lens)


# device time: 35807 ns/iter; 1.6944x vs baseline; 1.0102x over previous
import jax
import jax.numpy as jnp
from jax import lax
from jax.experimental import pallas as pl
from jax.experimental.pallas import tpu as pltpu

N_DEV = 8
B = 8
H = 8
D = 128
BS = 16
PAGES = 512
NB = 512
NK = PAGES * BS
CP = 64
C = PAGES // CP
CK = CP * BS
NEG = -1e30
SCALE = D ** -0.5


def kernel(Q, K, V, bt, lens):
    qT = Q.reshape(B, H, D).transpose(1, 0, 2)

    def body(q_ref, k_hbm, v_hbm, bt_ref, lens_ref, out_ref,
             w_ref, m_run, l_run, acc_run, kbuf, vbuf, comm_acc, comm_ml,
             ksem, vsem, acc_send, acc_recv, ml_send, ml_recv):
        c = pl.program_id(0)
        my = lax.axis_index("i")
        base = my * PAGES
        barrier = pltpu.get_barrier_semaphore()

        def kv_dma(cc, slot):
            ck = pltpu.make_async_copy(
                k_hbm.at[pl.ds(cc * CP, CP)], kbuf.at[slot], ksem.at[slot])
            cv = pltpu.make_async_copy(
                v_hbm.at[pl.ds(cc * CP, CP)], vbuf.at[slot], vsem.at[slot])
            return ck, cv

        @pl.when(c == 0)
        def _first():
            for cc in range(2):
                ck, cv = kv_dma(cc, cc)
                ck.start()
                cv.start()
            for tgt in range(N_DEV):
                @pl.when(my != tgt)
                def _(tgt=tgt):
                    pl.semaphore_signal(
                        barrier, inc=1,
                        device_id=(tgt,), device_id_type=pl.DeviceIdType.MESH,
                    )
            m_run[...] = jnp.full((H * B, 1), NEG, jnp.float32)
            l_run[...] = jnp.zeros((H * B, 1), jnp.float32)
            acc_run[...] = jnp.zeros((H * B, D), jnp.float32)
            btT = jnp.transpose(bt_ref[...])
            p_row = lax.broadcasted_iota(jnp.int32, (NB, PAGES), 1)
            j_col = lax.broadcasted_iota(jnp.int32, (NB, 1), 0)
            rows = []
            for b in range(B):
                col = btT[:, b:b + 1]
                valid = j_col < lens_ref[b]
                match = (col == base + p_row) & valid
                rows.append(jnp.sum(
                    jnp.where(match, 1.0, 0.0).astype(jnp.float32),
                    axis=0, keepdims=True))
            cnt = jnp.concatenate(rows, axis=0)
            k16 = lax.broadcasted_iota(jnp.int32, (PAGES, NK), 1) // BS
            p_col = lax.broadcasted_iota(jnp.int32, (PAGES, NK), 0)
            expand = jnp.where(k16 == p_col, 1.0, 0.0).astype(jnp.bfloat16)
            w_ref[...] = lax.dot_general(
                cnt.astype(jnp.bfloat16), expand, (((1,), (0,)), ((), ())),
                preferred_element_type=jnp.float32)

        @pl.when(c + 2 < C)
        def _prefetch():
            ck, cv = kv_dma(c + 2, lax.rem(c + 2, 3))
            ck.start()
            cv.start()

        slot = lax.rem(c, 3)
        ck, cv = kv_dma(c, slot)
        ck.wait()
        kflat = kbuf[slot].reshape(CK, H * D)
        q3 = q_ref[...]
        wc = w_ref[:, pl.ds(c * CK, CK)]
        s_all = jnp.concatenate(
            [lax.dot_general(q3[h], kflat[:, h * D:(h + 1) * D],
                             (((1,), (1,)), ((), ())),
                             preferred_element_type=jnp.float32)
             for h in range(H)], axis=0)
        wt = jnp.broadcast_to(wc[None], (H, B, CK)).reshape(H * B, CK)
        s_all = jnp.where(wt > 0, s_all * SCALE, NEG)
        m_c = jnp.max(s_all, axis=1, keepdims=True)
        m_new = jnp.maximum(m_run[...], m_c)
        alpha = jnp.exp(m_run[...] - m_new)
        p = jnp.exp(s_all - m_new) * wt
        cv.wait()
        vflat = vbuf[slot].reshape(CK, H * D)
        pv = jnp.concatenate(
            [lax.dot_general(p[h * B:(h + 1) * B],
                             vflat[:, h * D:(h + 1) * D],
                             (((1,), (0,)), ((), ())),
                             preferred_element_type=jnp.float32)
             for h in range(H)], axis=0)
        l_run[...] = l_run[...] * alpha + jnp.sum(p, axis=1, keepdims=True)
        acc_run[...] = acc_run[...] * alpha + pv
        m_run[...] = m_new

        def acc_rdma(slot_, tgt):
            return pltpu.make_async_remote_copy(
                src_ref=comm_acc.at[slot_], dst_ref=comm_acc.at[slot_],
                send_sem=acc_send.at[tgt], recv_sem=acc_recv.at[slot_],
                device_id=(tgt,), device_id_type=pl.DeviceIdType.MESH)

        def ml_rdma(slot_, tgt):
            return pltpu.make_async_remote_copy(
                src_ref=comm_ml.at[slot_], dst_ref=comm_ml.at[slot_],
                send_sem=ml_send.at[tgt], recv_sem=ml_recv.at[slot_],
                device_id=(tgt,), device_id_type=pl.DeviceIdType.MESH)

        @pl.when(c == C - 1)
        def _last():
            comm_acc[my] = acc_run[...].reshape(H, B, D)
            comm_ml[my, :, 0] = m_run[...].reshape(H, B, 1)
            comm_ml[my, :, 1] = l_run[...].reshape(H, B, 1)
            pl.semaphore_wait(barrier, N_DEV - 1)
            for tgt in range(N_DEV):
                @pl.when(my != tgt)
                def _(tgt=tgt):
                    acc_rdma(my, tgt).start()
                    ml_rdma(my, tgt).start()
            for src in range(N_DEV):
                @pl.when(my != src)
                def _(src=src):
                    acc_rdma(src, src).wait_recv()
                    ml_rdma(src, src).wait_recv()
            for tgt in range(N_DEV):
                @pl.when(my != tgt)
                def _(tgt=tgt):
                    acc_rdma(my, tgt).wait_send()
                    ml_rdma(my, tgt).wait_send()

            A = comm_acc[...]
            ml = comm_ml[...]
            m_all = ml[:, :, 0, :, 0]
            l_all = ml[:, :, 1, :, 0]
            M = jnp.max(m_all, axis=0, keepdims=True)
            w_dev = jnp.exp(m_all - M)
            num = jnp.sum(A * w_dev[..., None], axis=0)
            den = jnp.sum(l_all * w_dev, axis=0)
            out = (num / den[..., None]).transpose(1, 0, 2)
            out_ref[...] = out[:, None, :, :]

    return pl.pallas_call(
        body,
        grid=(C,),
        out_shape=jax.ShapeDtypeStruct((B, 1, H, D), jnp.float32),
        in_specs=[
            pl.BlockSpec(memory_space=pltpu.MemorySpace.VMEM),
            pl.BlockSpec(memory_space=pltpu.MemorySpace.HBM),
            pl.BlockSpec(memory_space=pltpu.MemorySpace.HBM),
            pl.BlockSpec(memory_space=pltpu.MemorySpace.VMEM),
            pl.BlockSpec(memory_space=pltpu.SMEM),
        ],
        out_specs=pl.BlockSpec(memory_space=pltpu.MemorySpace.VMEM),
        scratch_shapes=[
            pltpu.VMEM((B, NK), jnp.float32),
            pltpu.VMEM((H * B, 1), jnp.float32),
            pltpu.VMEM((H * B, 1), jnp.float32),
            pltpu.VMEM((H * B, D), jnp.float32),
            pltpu.VMEM((3, CP, BS, H, D), jnp.float32),
            pltpu.VMEM((3, CP, BS, H, D), jnp.float32),
            pltpu.VMEM((N_DEV, H, B, D), jnp.float32),
            pltpu.VMEM((N_DEV, H, 2, B, 1), jnp.float32),
            pltpu.SemaphoreType.DMA((3,)),
            pltpu.SemaphoreType.DMA((3,)),
            pltpu.SemaphoreType.DMA((N_DEV,)),
            pltpu.SemaphoreType.DMA((N_DEV,)),
            pltpu.SemaphoreType.DMA((N_DEV,)),
            pltpu.SemaphoreType.DMA((N_DEV,)),
        ],
        compiler_params=pltpu.CompilerParams(
            collective_id=0,
            dimension_semantics=("arbitrary",),
        ),
    )(qT, K, V, bt, lens)


# device time: 35555 ns/iter; 1.7064x vs baseline; 1.0071x over previous
import jax
import jax.numpy as jnp
from jax import lax
from jax.experimental import pallas as pl
from jax.experimental.pallas import tpu as pltpu

N_DEV = 8
B = 8
H = 8
D = 128
BS = 16
PAGES = 512
NB = 512
NK = PAGES * BS
CP = 64
C = PAGES // CP
CK = CP * BS
NEG = -1e30
SCALE = D ** -0.5


def kernel(Q, K, V, bt, lens):
    qT = Q.reshape(B, H, D).transpose(1, 0, 2)

    def body(q_ref, k_hbm, v_hbm, bt_ref, lens_ref, out_ref,
             w_ref, qw_ref, m_run, l_run, acc_run, kbuf, vbuf,
             comm_acc, comm_ml,
             ksem, vsem, acc_send, acc_recv, ml_send, ml_recv):
        c = pl.program_id(0)
        my = lax.axis_index("i")
        base = my * PAGES
        barrier = pltpu.get_barrier_semaphore()

        def kv_dma(cc, slot):
            ck = pltpu.make_async_copy(
                k_hbm.at[pl.ds(cc * CP, CP)], kbuf.at[slot], ksem.at[slot])
            cv = pltpu.make_async_copy(
                v_hbm.at[pl.ds(cc * CP, CP)], vbuf.at[slot], vsem.at[slot])
            return ck, cv

        @pl.when(c == 0)
        def _first():
            for cc in range(2):
                ck, cv = kv_dma(cc, cc)
                ck.start()
                cv.start()
            for tgt in range(N_DEV):
                @pl.when(my != tgt)
                def _(tgt=tgt):
                    pl.semaphore_signal(
                        barrier, inc=1,
                        device_id=(tgt,), device_id_type=pl.DeviceIdType.MESH,
                    )
            m_run[...] = jnp.full((H * B, 1), NEG, jnp.float32)
            l_run[...] = jnp.zeros((H * B, 1), jnp.float32)
            acc_run[...] = jnp.zeros((H * B, D), jnp.float32)
            qflat = q_ref[...].reshape(H * B, D)
            q_tiled = jnp.concatenate([qflat] * H, axis=1)
            rowh = lax.broadcasted_iota(jnp.int32, (H * B, H * D), 0) // B
            colh = lax.broadcasted_iota(jnp.int32, (H * B, H * D), 1) // D
            qw_ref[...] = jnp.where(rowh == colh, q_tiled, 0.0)
            btT = jnp.transpose(bt_ref[...])
            p_row = lax.broadcasted_iota(jnp.int32, (NB, PAGES), 1)
            j_col = lax.broadcasted_iota(jnp.int32, (NB, 1), 0)
            rows = []
            for b in range(B):
                col = btT[:, b:b + 1]
                valid = j_col < lens_ref[b]
                match = (col == base + p_row) & valid
                rows.append(jnp.sum(
                    jnp.where(match, 1.0, 0.0).astype(jnp.float32),
                    axis=0, keepdims=True))
            cnt = jnp.concatenate(rows, axis=0)
            k16 = lax.broadcasted_iota(jnp.int32, (PAGES, NK), 1) // BS
            p_col = lax.broadcasted_iota(jnp.int32, (PAGES, NK), 0)
            expand = jnp.where(k16 == p_col, 1.0, 0.0).astype(jnp.bfloat16)
            w_ref[...] = lax.dot_general(
                cnt.astype(jnp.bfloat16), expand, (((1,), (0,)), ((), ())),
                preferred_element_type=jnp.float32)

        @pl.when(c + 2 < C)
        def _prefetch():
            ck, cv = kv_dma(c + 2, lax.rem(c + 2, 3))
            ck.start()
            cv.start()

        slot = lax.rem(c, 3)
        ck, cv = kv_dma(c, slot)
        ck.wait()
        kflat = kbuf[slot].reshape(CK, H * D)
        wc = w_ref[:, pl.ds(c * CK, CK)]
        s_all = lax.dot_general(qw_ref[...], kflat,
                                (((1,), (1,)), ((), ())),
                                preferred_element_type=jnp.float32)
        wt = jnp.broadcast_to(wc[None], (H, B, CK)).reshape(H * B, CK)
        s_all = jnp.where(wt > 0, s_all * SCALE, NEG)
        m_c = jnp.max(s_all, axis=1, keepdims=True)
        m_new = jnp.maximum(m_run[...], m_c)
        alpha = jnp.exp(m_run[...] - m_new)
        p = jnp.exp(s_all - m_new) * wt
        cv.wait()
        vflat = vbuf[slot].reshape(CK, H * D)
        pv_full = lax.dot_general(p, vflat, (((1,), (0,)), ((), ())),
                                  preferred_element_type=jnp.float32)
        pv = jnp.concatenate(
            [pv_full[h * B:(h + 1) * B, h * D:(h + 1) * D]
             for h in range(H)], axis=0)
        l_run[...] = l_run[...] * alpha + jnp.sum(p, axis=1, keepdims=True)
        acc_run[...] = acc_run[...] * alpha + pv
        m_run[...] = m_new

        def acc_rdma(slot_, tgt):
            return pltpu.make_async_remote_copy(
                src_ref=comm_acc.at[slot_], dst_ref=comm_acc.at[slot_],
                send_sem=acc_send.at[tgt], recv_sem=acc_recv.at[slot_],
                device_id=(tgt,), device_id_type=pl.DeviceIdType.MESH)

        def ml_rdma(slot_, tgt):
            return pltpu.make_async_remote_copy(
                src_ref=comm_ml.at[slot_], dst_ref=comm_ml.at[slot_],
                send_sem=ml_send.at[tgt], recv_sem=ml_recv.at[slot_],
                device_id=(tgt,), device_id_type=pl.DeviceIdType.MESH)

        @pl.when(c == C - 1)
        def _last():
            comm_acc[my] = acc_run[...].reshape(H, B, D)
            comm_ml[my, :, 0] = m_run[...].reshape(H, B, 1)
            comm_ml[my, :, 1] = l_run[...].reshape(H, B, 1)
            pl.semaphore_wait(barrier, N_DEV - 1)
            for tgt in range(N_DEV):
                @pl.when(my != tgt)
                def _(tgt=tgt):
                    acc_rdma(my, tgt).start()
                    ml_rdma(my, tgt).start()
            for src in range(N_DEV):
                @pl.when(my != src)
                def _(src=src):
                    acc_rdma(src, src).wait_recv()
                    ml_rdma(src, src).wait_recv()
            for tgt in range(N_DEV):
                @pl.when(my != tgt)
                def _(tgt=tgt):
                    acc_rdma(my, tgt).wait_send()
                    ml_rdma(my, tgt).wait_send()

            A = comm_acc[...]
            ml = comm_ml[...]
            m_all = ml[:, :, 0, :, 0]
            l_all = ml[:, :, 1, :, 0]
            M = jnp.max(m_all, axis=0, keepdims=True)
            w_dev = jnp.exp(m_all - M)
            num = jnp.sum(A * w_dev[..., None], axis=0)
            den = jnp.sum(l_all * w_dev, axis=0)
            out = (num / den[..., None]).transpose(1, 0, 2)
            out_ref[...] = out[:, None, :, :]

    return pl.pallas_call(
        body,
        grid=(C,),
        out_shape=jax.ShapeDtypeStruct((B, 1, H, D), jnp.float32),
        in_specs=[
            pl.BlockSpec(memory_space=pltpu.MemorySpace.VMEM),
            pl.BlockSpec(memory_space=pltpu.MemorySpace.HBM),
            pl.BlockSpec(memory_space=pltpu.MemorySpace.HBM),
            pl.BlockSpec(memory_space=pltpu.MemorySpace.VMEM),
            pl.BlockSpec(memory_space=pltpu.SMEM),
        ],
        out_specs=pl.BlockSpec(memory_space=pltpu.MemorySpace.VMEM),
        scratch_shapes=[
            pltpu.VMEM((B, NK), jnp.float32),
            pltpu.VMEM((H * B, H * D), jnp.float32),
            pltpu.VMEM((H * B, 1), jnp.float32),
            pltpu.VMEM((H * B, 1), jnp.float32),
            pltpu.VMEM((H * B, D), jnp.float32),
            pltpu.VMEM((3, CP, BS, H, D), jnp.float32),
            pltpu.VMEM((3, CP, BS, H, D), jnp.float32),
            pltpu.VMEM((N_DEV, H, B, D), jnp.float32),
            pltpu.VMEM((N_DEV, H, 2, B, 1), jnp.float32),
            pltpu.SemaphoreType.DMA((3,)),
            pltpu.SemaphoreType.DMA((3,)),
            pltpu.SemaphoreType.DMA((N_DEV,)),
            pltpu.SemaphoreType.DMA((N_DEV,)),
            pltpu.SemaphoreType.DMA((N_DEV,)),
            pltpu.SemaphoreType.DMA((N_DEV,)),
        ],
        compiler_params=pltpu.CompilerParams(
            collective_id=0,
            dimension_semantics=("arbitrary",),
        ),
    )(qT, K, V, bt, lens)
